# baseline (speedup 1.0000x reference)
import jax
import jax.numpy as jnp
import numpy as np
from jax.experimental import pallas as pl
from jax.experimental.pallas import tpu as pltpu

N_T, N_Y, N_X = 8, 32, 32
NB = N_X * N_Y
OFFS = (-64, -33, -32, -31, -2, -1, 0, 1, 2, 31, 32, 33, 64)
S = (-32, -1, 0, 1, 32)
N_BLK = 3 * N_T - 2
ROW_TILE = 512


def _np_masks():
    k = np.arange(NB)
    x = k % N_X
    y = k // N_X
    me = ((x + 1) < N_X).astype(np.float32)
    mw = ((x - 1) >= 0).astype(np.float32)
    mn = ((y + 1) < N_Y).astype(np.float32)
    ms = ((y - 1) >= 0).astype(np.float32)
    return me, mw, mn, ms


_ME, _MW, _MN, _MS = _np_masks()


def _shift_lanes(v, o):
    if o == 0:
        return v
    z = jnp.zeros((1, abs(o)), v.dtype)
    if o > 0:
        return jnp.concatenate([z, v[:, : NB - o]], axis=1)
    return jnp.concatenate([v[:, -o:], z], axis=1)


SUB = 128


def _band_kernel(a_ref, b_ref, w_ref, e_ref, out_ref):
    w = w_ref[0, 0]
    dd = {d: None for d in OFFS}
    for i1, o1 in enumerate(S):
        aw = a_ref[0, 0, i1 : i1 + 1, :] * w
        for i2, o2 in enumerate(S):
            term = _shift_lanes(aw * b_ref[0, 0, i2 : i2 + 1, :], o1)
            d = o2 - o1
            dd[d] = term if dd[d] is None else dd[d] + term
    dd[0] = dd[0] + e_ref[0, 0]
    g = {d: _shift_lanes(dd[d], d) for d in OFFS}
    nsub = NB // SUB
    jr = jax.lax.broadcasted_iota(jnp.int32, (SUB, SUB), 1) - jax.lax.broadcasted_iota(
        jnp.int32, (SUB, SUB), 0
    )
    masks = {}
    for delta in (-1, 0, 1):
        for d in OFFS:
            c = d - SUB * delta
            if -SUB < c < SUB:
                masks[(delta, d)] = jr == c
    zero = jnp.zeros((SUB, SUB), jnp.float32)
    for sr in range(nsub):
        for sc in range(nsub):
            delta = sc - sr
            if abs(delta) > 1:
                out_ref[0, 0, sr * SUB : (sr + 1) * SUB, sc * SUB : (sc + 1) * SUB] = zero
                continue
            acc = zero
            for d in OFFS:
                if (delta, d) not in masks:
                    continue
                gd = g[d][:, sc * SUB : (sc + 1) * SUB]
                acc = jnp.where(masks[(delta, d)], jnp.broadcast_to(gd, (SUB, SUB)), acc)
            out_ref[0, 0, sr * SUB : (sr + 1) * SUB, sc * SUB : (sc + 1) * SUB] = acc


def kernel(kappa, m, H, tau):
    del H
    kap = kappa[0]
    t = jnp.squeeze(tau, axis=1)
    qt = jnp.transpose(1.0 / (t * t), (0, 2, 1))
    m1 = jnp.transpose(m[:, 0], (0, 2, 1))
    m2 = jnp.transpose(m[:, 1], (0, 2, 1))
    u1 = 0.5 * m1 * _ME
    l1 = -0.5 * m1 * _MW
    u32 = 0.5 * m2 * _MN
    l32 = -0.5 * m2 * _MS
    k2 = kap * kap
    dvec = jnp.concatenate(
        [jnp.full((2, 1, NB), k2), jnp.full((2, N_T - 1, NB), 1.0 + k2)], axis=1
    )
    Md = jnp.stack([l32, l1, dvec, u1, u32], axis=2)

    e0 = jnp.zeros((2, 1, 5, NB), jnp.float32).at[:, :, 2, :].set(1.0)
    Md_ext = jnp.concatenate([Md, e0], axis=1)
    qt_ext = jnp.concatenate([qt, jnp.ones((2, 1, NB), jnp.float32)], axis=1)

    KA, KB, KW, SW, KE, SE = [0, 8], [0, 1], [8, 1], [1.0, -1.0], [8, 8], [1.05, 0.0]
    for i in range(1, N_T - 1):
        KA += [i, i, 8]
        KB += [8, i, i + 1]
        KW += [i, i, i + 1]
        SW += [-1.0, 1.0, -1.0]
        KE += [8, i, 8]
        SE += [0.0, 1.0, 0.0]
    KA += [N_T - 1, N_T - 1]
    KB += [8, N_T - 1]
    KW += [N_T - 1, N_T - 1]
    SW += [-1.0, 1.0]
    KE += [8, 8]
    SE += [0.0, 0.0]

    sw = jnp.asarray(np.asarray(SW, np.float32))[None, :, None]
    se = jnp.asarray(np.asarray(SE, np.float32))[None, :, None]
    A = jnp.take(Md_ext, jnp.asarray(KA), axis=1)
    B = jnp.take(Md_ext, jnp.asarray(KB), axis=1)
    W = (jnp.take(qt_ext, jnp.asarray(KW), axis=1) * sw)[:, :, None, :]
    E = (jnp.take(qt_ext, jnp.asarray(KE), axis=1) * se)[:, :, None, :]

    return pl.pallas_call(
        _band_kernel,
        grid=(2, N_BLK),
        in_specs=[
            pl.BlockSpec((1, 1, 5, NB), lambda b, k: (b, k, 0, 0)),
            pl.BlockSpec((1, 1, 5, NB), lambda b, k: (b, k, 0, 0)),
            pl.BlockSpec((1, 1, 1, NB), lambda b, k: (b, k, 0, 0)),
            pl.BlockSpec((1, 1, 1, NB), lambda b, k: (b, k, 0, 0)),
        ],
        out_specs=pl.BlockSpec((1, 1, NB, NB), lambda b, k: (b, k, 0, 0)),
        out_shape=jax.ShapeDtypeStruct((2, N_BLK, NB, NB), jnp.float32),
        compiler_params=pltpu.CompilerParams(
            dimension_semantics=("parallel", "parallel")
        ),
    )(A, B, W, E)
